# baseline (speedup 1.0000x reference)
import jax
import jax.numpy as jnp
from jax import lax
from jax.experimental import pallas as pl
from jax.experimental.pallas import tpu as pltpu
from jax.experimental.pallas import tpu_sc as plsc

NUM_OPS = 100000
EMBED_D = 64
N_NODES = 16384

_info = plsc.get_sparse_core_info()
_NC = _info.num_cores
_NS = _info.num_subcores
_NW = _NC * _NS
_DPW = EMBED_D // _NW
_SEGA = 49920
_SEGB = 50048
_TAIL = _SEGA + _SEGB
_Q = N_NODES // 4
_GRP = _Q // 16


def _gather_body(idx_hbm, table_hbm, out_hbm, row_v, idx_v, tail_v, out_a,
                 out_b, out_c, isem, rsa, rsb, rst, ws0, ws1, ws2):
    wid = lax.axis_index("s") * _NC + lax.axis_index("c")
    c0 = wid * _DPW
    iota = lax.iota(jnp.int32, 16)
    obufs = (out_a, out_b, out_c)
    wsems = (ws0, ws1, ws2)

    def fire_a(c):
        return pltpu.async_copy(
            table_hbm.at[c, pl.ds(0, _SEGA)], row_v.at[pl.ds(0, _SEGA)], rsa
        )

    def fire_bt(c):
        return (
            pltpu.async_copy(
                table_hbm.at[c, pl.ds(_SEGA, _SEGB)],
                row_v.at[pl.ds(_SEGA, _SEGB)],
                rsb,
            ),
            pltpu.async_copy(
                table_hbm.at[c, pl.ds(_TAIL, NUM_OPS - _TAIL)], tail_v, rst
            ),
        )

    rcpa = fire_a(c0)
    rcpb, rcpt = fire_bt(c0)
    icp = pltpu.async_copy(idx_hbm, idx_v, isem)
    pending = [None, None, None]

    def t0_scan(q, b):
        if pending[b] is not None:
            pending[b].wait()
        ov = obufs[b]
        qb = q * _Q

        @plsc.parallel_loop(0, _GRP, unroll=16)
        def _(g, _ov=ov, _qb=qb):
            iv = idx_v[pl.ds(_qb + g * 16, 16)]
            _ov[pl.ds(g * 16, 16)] = plsc.load_gather(row_v, [iv])

    def t1_scan_flush(q, b, c):
        ov = obufs[b]
        qb = q * _Q

        @plsc.parallel_loop(0, _GRP, unroll=16)
        def _(g, _ov=ov, _qb=qb):
            iv = idx_v[pl.ds(_qb + g * 16, 16)]
            m = iv >= jnp.int32(_SEGA)
            v = plsc.load_gather(row_v, [iv])
            plsc.store_scatter(_ov, [g * 16 + iota], v, mask=m)

        pending[b] = pltpu.async_copy(
            ov, out_hbm.at[c, pl.ds(qb, _Q)], wsems[b]
        )

    for d in range(_DPW):
        c = c0 + d
        rcpa.wait()
        if d == 0:
            icp.wait()
        t0_scan(0, 0)
        t0_scan(1, 1)
        rcpb.wait()
        rcpt.wait()
        for _j in range(2):
            row_v[pl.ds(_TAIL + _j * 16, 16)] = tail_v[pl.ds(_j * 16, 16)]
        t1_scan_flush(0, 0, c)
        t1_scan_flush(1, 1, c)
        t0_scan(2, 2)
        t1_scan_flush(2, 2, c)
        t0_scan(3, 0)
        if d + 1 < _DPW:
            rcpa = fire_a(c + 1)
        t1_scan_flush(3, 0, c)
        if d + 1 < _DPW:
            rcpb, rcpt = fire_bt(c + 1)
    for b in range(3):
        if pending[b] is not None:
            pending[b].wait()


@jax.jit
def kernel(op_ids, table):
    mesh = plsc.VectorSubcoreMesh(core_axis_name="c", subcore_axis_name="s")
    f = pl.kernel(
        _gather_body,
        out_type=jax.ShapeDtypeStruct((EMBED_D, N_NODES), jnp.float32),
        mesh=mesh,
        scratch_types=[
            pltpu.VMEM((NUM_OPS,), jnp.float32),
            pltpu.VMEM((N_NODES,), jnp.int32),
            pltpu.VMEM((NUM_OPS - _TAIL,), jnp.float32),
            pltpu.VMEM((_Q,), jnp.float32),
            pltpu.VMEM((_Q,), jnp.float32),
            pltpu.VMEM((_Q,), jnp.float32),
            pltpu.SemaphoreType.DMA,
            pltpu.SemaphoreType.DMA,
            pltpu.SemaphoreType.DMA,
            pltpu.SemaphoreType.DMA,
            pltpu.SemaphoreType.DMA,
            pltpu.SemaphoreType.DMA,
            pltpu.SemaphoreType.DMA,
        ],
        compiler_params=pltpu.CompilerParams(needs_layout_passes=False),
    )
    out_t = f(op_ids.astype(jnp.int32), table.T)
    return out_t.T


# speedup vs baseline: 1.0946x; 1.0946x over previous
import jax
import jax.numpy as jnp
from jax import lax
from jax.experimental import pallas as pl
from jax.experimental.pallas import tpu as pltpu
from jax.experimental.pallas import tpu_sc as plsc

NUM_OPS = 100000
EMBED_D = 64
N_NODES = 16384

_info = plsc.get_sparse_core_info()
_NC = _info.num_cores
_NS = _info.num_subcores
_NW = _NC * _NS
_DPW = EMBED_D // _NW
_SEGA = 49920
_SEGB = 50048
_TAIL = _SEGA + _SEGB
_Q = N_NODES // 4
_GRP = _Q // 16


def _gather_body(idx_hbm, table_hbm, out_hbm, row_v, idx_v, tail_v, out_a,
                 out_b, isem, rsa, rsb, rst, wsem_a, wsem_b):
    wid = lax.axis_index("s") * _NC + lax.axis_index("c")
    c0 = wid * _DPW

    def fire_row(c):
        return (
            pltpu.async_copy(
                table_hbm.at[c, pl.ds(0, _SEGA)],
                row_v.at[pl.ds(0, _SEGA)], rsa,
            ),
            pltpu.async_copy(
                table_hbm.at[c, pl.ds(_SEGA, _SEGB)],
                row_v.at[pl.ds(_SEGA, _SEGB)], rsb,
            ),
            pltpu.async_copy(
                table_hbm.at[c, pl.ds(_TAIL, NUM_OPS - _TAIL)], tail_v, rst
            ),
        )

    rcps = fire_row(c0)
    pltpu.sync_copy(idx_hbm, idx_v)
    obufs = (out_a, out_b)
    wsems = (wsem_a, wsem_b)
    pending = [None, None]
    for d in range(_DPW):
        c = c0 + d
        for rcp in rcps:
            rcp.wait()
        for _j in range(2):
            row_v[pl.ds(_TAIL + _j * 16, 16)] = tail_v[pl.ds(_j * 16, 16)]
        for q in range(4):
            b = q % 2
            if pending[b] is not None:
                pending[b].wait()
            ov = obufs[b]
            qbase = q * _Q

            @plsc.parallel_loop(0, _GRP, unroll=16)
            def _(g, _ov=ov, _qb=qbase):
                iv = idx_v[pl.ds(_qb + g * 16, 16)]
                _ov[pl.ds(g * 16, 16)] = plsc.load_gather(row_v, [iv])

            if q == 3 and d + 1 < _DPW:
                rcps = fire_row(c + 1)
            pending[b] = pltpu.async_copy(
                ov, out_hbm.at[c, pl.ds(qbase, _Q)], wsems[b]
            )
    for b in range(2):
        if pending[b] is not None:
            pending[b].wait()


@jax.jit
def kernel(op_ids, table):
    mesh = plsc.VectorSubcoreMesh(core_axis_name="c", subcore_axis_name="s")
    f = pl.kernel(
        _gather_body,
        out_type=jax.ShapeDtypeStruct((EMBED_D, N_NODES), jnp.float32),
        mesh=mesh,
        scratch_types=[
            pltpu.VMEM((NUM_OPS,), jnp.float32),
            pltpu.VMEM((N_NODES,), jnp.int32),
            pltpu.VMEM((NUM_OPS - _TAIL,), jnp.float32),
            pltpu.VMEM((_Q,), jnp.float32),
            pltpu.VMEM((_Q,), jnp.float32),
            pltpu.SemaphoreType.DMA,
            pltpu.SemaphoreType.DMA,
            pltpu.SemaphoreType.DMA,
            pltpu.SemaphoreType.DMA,
            pltpu.SemaphoreType.DMA,
            pltpu.SemaphoreType.DMA,
        ],
        compiler_params=pltpu.CompilerParams(needs_layout_passes=False),
    )
    out_t = f(op_ids.astype(jnp.int32), table.T)
    return out_t.T


# speedup vs baseline: 1.1078x; 1.0120x over previous
import jax
import jax.numpy as jnp
from jax import lax
from jax.experimental import pallas as pl
from jax.experimental.pallas import tpu as pltpu
from jax.experimental.pallas import tpu_sc as plsc

NUM_OPS = 100000
EMBED_D = 64
N_NODES = 16384

_info = plsc.get_sparse_core_info()
_NC = _info.num_cores
_NS = _info.num_subcores
_NW = _NC * _NS
_DPW = EMBED_D // _NW
_Q = N_NODES // 4
_GRP = _Q // 16


def _gather_body(idx_hbm, table_hbm, out_hbm, row_v, idx_v, out_a, out_b,
                 rsem, wsem_a, wsem_b):
    wid = lax.axis_index("s") * _NC + lax.axis_index("c")
    c0 = wid * _DPW
    rcp = pltpu.async_copy(table_hbm.at[c0], row_v, rsem)
    pltpu.sync_copy(idx_hbm, idx_v)
    obufs = (out_a, out_b)
    wsems = (wsem_a, wsem_b)
    pending = [None, None]
    for d in range(_DPW):
        c = c0 + d
        rcp.wait()
        for q in range(4):
            b = q % 2
            if pending[b] is not None:
                pending[b].wait()
            ov = obufs[b]
            qbase = q * _Q

            @plsc.parallel_loop(0, _GRP, unroll=16)
            def _(g, _ov=ov, _qb=qbase):
                iv = idx_v[pl.ds(_qb + g * 16, 16)]
                _ov[pl.ds(g * 16, 16)] = plsc.load_gather(row_v, [iv])

            if q == 3 and d + 1 < _DPW:
                rcp = pltpu.async_copy(table_hbm.at[c + 1], row_v, rsem)
            pending[b] = pltpu.async_copy(
                ov, out_hbm.at[c, pl.ds(qbase, _Q)], wsems[b]
            )
    for b in range(2):
        if pending[b] is not None:
            pending[b].wait()


@jax.jit
def kernel(op_ids, table):
    mesh = plsc.VectorSubcoreMesh(core_axis_name="c", subcore_axis_name="s")
    f = pl.kernel(
        _gather_body,
        out_type=jax.ShapeDtypeStruct((EMBED_D, N_NODES), jnp.float32),
        mesh=mesh,
        scratch_types=[
            pltpu.VMEM((NUM_OPS,), jnp.float32),
            pltpu.VMEM((N_NODES,), jnp.int32),
            pltpu.VMEM((_Q,), jnp.float32),
            pltpu.VMEM((_Q,), jnp.float32),
            pltpu.SemaphoreType.DMA,
            pltpu.SemaphoreType.DMA,
            pltpu.SemaphoreType.DMA,
        ],
        compiler_params=pltpu.CompilerParams(needs_layout_passes=False),
    )
    out_t = f(op_ids.astype(jnp.int32), table.T)
    return out_t.T
